# baseline (speedup 1.0000x reference)
import functools

import jax
import jax.numpy as jnp
from jax import lax
from jax.experimental import pallas as pl
from jax.experimental.pallas import tpu as pltpu
from jax.experimental.pallas import tpu_sc as plsc

DIM = 64
EPS = 1e-5

NC = 2
NS = 16
L = 16
NW = NC * NS

B_TOTAL = 4096 * 200
PER_W = B_TOTAL // NW
CHUNK = 256
N_CHUNKS = PER_W // CHUNK
GPC = CHUNK // L
ISTREAMS = CHUNK // 128
PAIR_W = 2 * DIM


def _rsqrt_nr(x):
    i = plsc.bitcast(x, jnp.int32)
    i = jnp.int32(0x5F3759DF) - (i >> 1)
    y = plsc.bitcast(i, jnp.float32)
    for _ in range(3):
        y = y * (1.5 - 0.5 * x * y * y)
    return y


@functools.cache
def _make_sc_kernel():
    mesh = plsc.VectorSubcoreMesh(
        core_axis_name="c", subcore_axis_name="s",
        num_cores=NC, num_subcores=NS)

    @functools.partial(
        pl.kernel,
        mesh=mesh,
        out_type=jax.ShapeDtypeStruct((B_TOTAL, DIM), jnp.float32),
        scratch_types=[
            pltpu.VMEM((ISTREAMS, 128), jnp.int32),
            pltpu.VMEM((ISTREAMS, 128), jnp.int32),
            pltpu.VMEM((CHUNK,), jnp.int32),
            pltpu.VMEM((CHUNK, PAIR_W), jnp.float32),
            pltpu.VMEM((CHUNK, DIM), jnp.float32),
            pltpu.VMEM((DIM,), jnp.float32),
            pltpu.VMEM((DIM,), jnp.float32),
            pltpu.SemaphoreType.DMA,
        ],
        compiler_params=pltpu.CompilerParams(needs_layout_passes=False),
    )
    def sc_kernel(x_hbm, table_hbm, gamma_hbm, beta_hbm, out_hbm,
                  idx_v, idxh_v, pb_v, rows_v, out_v, gam_v, bet_v, sem):
        wid = lax.axis_index("s") * NC + lax.axis_index("c")
        pltpu.sync_copy(gamma_hbm, gam_v)
        pltpu.sync_copy(beta_hbm, bet_v)
        idx_row0 = wid * (PER_W // 128)
        out_row0 = wid * PER_W

        @pl.loop(0, N_CHUNKS)
        def _chunk(i):
            pltpu.sync_copy(
                x_hbm.at[pl.ds(idx_row0 + i * ISTREAMS, ISTREAMS)], idx_v)
            for j in range(ISTREAMS):
                for k in range(128 // L):
                    v = idx_v[j, pl.ds(k * L, L)]
                    idxh_v[j, pl.ds(k * L, L)] = v >> 1
                    pb_v[pl.ds(j * 128 + k * L, L)] = (v & 1) * DIM
            for j in range(ISTREAMS):
                pltpu.async_copy(
                    table_hbm.at[idxh_v.at[j]],
                    rows_v.at[pl.ds(j * 128, 128)],
                    sem,
                ).wait()

            @pl.loop(0, GPC)
            def _group(g):
                riota = g * L + lax.iota(jnp.int32, L)
                pbase = pb_v[pl.ds(g * L, L)]
                s = jnp.zeros((L,), jnp.float32)
                q = jnp.zeros((L,), jnp.float32)
                for c in range(DIM):
                    col = plsc.load_gather(rows_v, [riota, pbase + c])
                    s = s + col
                    q = q + col * col
                mu = s * (1.0 / DIM)
                var = q * (1.0 / DIM) - mu * mu
                rs = _rsqrt_nr(var + EPS)
                for c in range(DIM):
                    col = plsc.load_gather(rows_v, [riota, pbase + c])
                    ci = jnp.full((L,), c, jnp.int32)
                    plsc.store_scatter(out_v, [riota, ci], (col - mu) * rs)
                gb = [(gam_v[pl.ds(k * L, L)], bet_v[pl.ds(k * L, L)])
                      for k in range(DIM // L)]
                for r in range(L):
                    row = g * L + r
                    for k, (gk, bk) in enumerate(gb):
                        v = out_v[row, pl.ds(k * L, L)]
                        out_v[row, pl.ds(k * L, L)] = v * gk + bk

            pltpu.sync_copy(
                out_v, out_hbm.at[pl.ds(out_row0 + i * CHUNK, CHUNK)])

    return sc_kernel


def kernel(x, table, gamma, beta):
    xf = x.reshape(B_TOTAL // 128, 128).astype(jnp.int32)
    table2 = table.reshape(table.shape[0] // 2, PAIR_W)
    out = _make_sc_kernel()(xf, table2, gamma, beta)
    return out.reshape(x.shape[0], x.shape[1], DIM)


# speedup vs baseline: 1.9146x; 1.9146x over previous
import functools

import jax
import jax.numpy as jnp
from jax import lax
from jax.experimental import pallas as pl
from jax.experimental.pallas import tpu as pltpu
from jax.experimental.pallas import tpu_sc as plsc

DIM = 64
EPS = 1e-5

NC = 2
NS = 16
L = 16
NW = NC * NS

B_TOTAL = 4096 * 200
PER_W = B_TOTAL // NW
CHUNK = 256
N_CHUNKS = PER_W // CHUNK
GPC = CHUNK // L
ISTREAMS = CHUNK // 128
PAIR_W = 2 * DIM


def _rsqrt_nr(x):
    i = plsc.bitcast(x, jnp.int32)
    i = jnp.int32(0x5F3759DF) - (i >> 1)
    y = plsc.bitcast(i, jnp.float32)
    for _ in range(3):
        y = y * (1.5 - 0.5 * x * y * y)
    return y


@functools.cache
def _make_sc_kernel():
    mesh = plsc.VectorSubcoreMesh(
        core_axis_name="c", subcore_axis_name="s",
        num_cores=NC, num_subcores=NS)

    @functools.partial(
        pl.kernel,
        mesh=mesh,
        out_type=jax.ShapeDtypeStruct((B_TOTAL, DIM), jnp.float32),
        scratch_types=[
            pltpu.VMEM((ISTREAMS, 128), jnp.int32),
            pltpu.VMEM((ISTREAMS, 128), jnp.int32),
            pltpu.VMEM((CHUNK,), jnp.int32),
            pltpu.VMEM((CHUNK, PAIR_W), jnp.float32),
            pltpu.VMEM((CHUNK, DIM), jnp.float32),
            pltpu.VMEM((DIM,), jnp.float32),
            pltpu.VMEM((DIM,), jnp.float32),
            pltpu.SemaphoreType.DMA,
        ],
        compiler_params=pltpu.CompilerParams(needs_layout_passes=False),
    )
    def sc_kernel(x_hbm, table_hbm, gamma_hbm, beta_hbm, out_hbm,
                  idx_v, idxh_v, pb_v, rows_v, out_v, gam_v, bet_v, sem):
        wid = lax.axis_index("s") * NC + lax.axis_index("c")
        pltpu.sync_copy(gamma_hbm, gam_v)
        pltpu.sync_copy(beta_hbm, bet_v)
        idx_row0 = wid * (PER_W // 128)
        out_row0 = wid * PER_W

        @pl.loop(0, N_CHUNKS)
        def _chunk(i):
            pltpu.sync_copy(
                x_hbm.at[pl.ds(idx_row0 + i * ISTREAMS, ISTREAMS)], idx_v)
            for j in range(ISTREAMS):
                for k in range(128 // L):
                    v = idx_v[j, pl.ds(k * L, L)]
                    idxh_v[j, pl.ds(k * L, L)] = v >> 1
                    pb_v[pl.ds(j * 128 + k * L, L)] = (v & 1) * DIM
            for j in range(ISTREAMS):
                pltpu.async_copy(
                    table_hbm.at[idxh_v.at[j]],
                    rows_v.at[pl.ds(j * 128, 128)],
                    sem,
                ).wait()

            @pl.loop(0, GPC)
            def _group(g):
                lane = lax.iota(jnp.int32, L)
                riota = g * L + lane
                pbase = pb_v[pl.ds(g * L, L)]
                s = jnp.zeros((L,), jnp.float32)
                q = jnp.zeros((L,), jnp.float32)
                for c in range(DIM):
                    rot = (lane + c) & (DIM - 1)
                    col = plsc.load_gather(rows_v, [riota, pbase + rot])
                    s = s + col
                    q = q + col * col
                mu = s * (1.0 / DIM)
                var = q * (1.0 / DIM) - mu * mu
                rs = _rsqrt_nr(var + EPS)
                for c in range(DIM):
                    rot = (lane + c) & (DIM - 1)
                    col = plsc.load_gather(rows_v, [riota, pbase + rot])
                    plsc.store_scatter(out_v, [riota, rot], (col - mu) * rs)
                gb = [(gam_v[pl.ds(k * L, L)], bet_v[pl.ds(k * L, L)])
                      for k in range(DIM // L)]
                for r in range(L):
                    row = g * L + r
                    for k, (gk, bk) in enumerate(gb):
                        v = out_v[row, pl.ds(k * L, L)]
                        out_v[row, pl.ds(k * L, L)] = v * gk + bk

            pltpu.sync_copy(
                out_v, out_hbm.at[pl.ds(out_row0 + i * CHUNK, CHUNK)])

    return sc_kernel


def kernel(x, table, gamma, beta):
    xf = x.reshape(B_TOTAL // 128, 128).astype(jnp.int32)
    table2 = table.reshape(table.shape[0] // 2, PAIR_W)
    out = _make_sc_kernel()(xf, table2, gamma, beta)
    return out.reshape(x.shape[0], x.shape[1], DIM)
